# baseline (speedup 1.0000x reference)
import functools

import jax
import jax.numpy as jnp
from jax import lax
from jax.experimental import pallas as pl
from jax.experimental.pallas import tpu as pltpu
from jax.experimental.pallas import tpu_sc as plsc

BATCH = 16384
HIST = 200
D = 64
B = BATCH * HIST

_info = plsc.get_sparse_core_info()
NC, NS = _info.num_cores, _info.num_subcores
NW = NC * NS
B_PER_W = B // NW
CHUNK = 512
N_CHUNKS = B_PER_W // CHUNK

assert B % (8 * NW) == 0
assert B_PER_W % CHUNK == 0


def _sc_gather(x_flat, emb):
    mesh = plsc.VectorSubcoreMesh(core_axis_name="c", subcore_axis_name="s")

    @functools.partial(
        pl.kernel,
        mesh=mesh,
        out_type=jax.ShapeDtypeStruct((B, D), jnp.float32),
        scratch_types=[
            pltpu.VMEM((CHUNK,), jnp.int32),
            pltpu.VMEM((CHUNK, D), jnp.float32),
            pltpu.SemaphoreType.DMA,
        ],
        compiler_params=pltpu.CompilerParams(use_tc_tiling_on_sc=False),
    )
    def body(x_hbm, emb_hbm, out_hbm, idx_v, rows_v, sem):
        wid = lax.axis_index("s") * NC + lax.axis_index("c")
        base = wid * B_PER_W

        def chunk_body(i, carry):
            off = base + i * CHUNK
            pltpu.sync_copy(x_hbm.at[pl.ds(off, CHUNK)], idx_v)
            pltpu.async_copy(emb_hbm.at[idx_v], rows_v, sem).wait()
            pltpu.sync_copy(rows_v, out_hbm.at[pl.ds(off, CHUNK)])
            return carry

        lax.fori_loop(0, N_CHUNKS, chunk_body, 0, unroll=False)

    return body(x_flat, emb)


def kernel(x, emb):
    x_flat = x.reshape(B).astype(jnp.int32)
    out = _sc_gather(x_flat, emb)
    return out.reshape(BATCH, HIST, D)


# speedup vs baseline: 1.0735x; 1.0735x over previous
import functools

import jax
import jax.numpy as jnp
from jax import lax
from jax.experimental import pallas as pl
from jax.experimental.pallas import tpu as pltpu
from jax.experimental.pallas import tpu_sc as plsc

BATCH = 16384
HIST = 200
D = 64
B = BATCH * HIST

_info = plsc.get_sparse_core_info()
NC, NS = _info.num_cores, _info.num_subcores
NW = NC * NS
B_PER_W = B // NW
CHUNK = 512
N_CHUNKS = B_PER_W // CHUNK

assert B % (8 * NW) == 0
assert B_PER_W % CHUNK == 0
assert N_CHUNKS % 2 == 0 and N_CHUNKS >= 4


def _sc_gather(x_flat, emb):
    mesh = plsc.VectorSubcoreMesh(core_axis_name="c", subcore_axis_name="s")

    @functools.partial(
        pl.kernel,
        mesh=mesh,
        out_type=jax.ShapeDtypeStruct((B, D), jnp.float32),
        scratch_types=[
            pltpu.VMEM((CHUNK,), jnp.int32),
            pltpu.VMEM((CHUNK,), jnp.int32),
            pltpu.VMEM((CHUNK, D), jnp.float32),
            pltpu.VMEM((CHUNK, D), jnp.float32),
            pltpu.SemaphoreType.DMA,
            pltpu.SemaphoreType.DMA,
            pltpu.SemaphoreType.DMA,
            pltpu.SemaphoreType.DMA,
            pltpu.SemaphoreType.DMA,
        ],
        compiler_params=pltpu.CompilerParams(use_tc_tiling_on_sc=False),
    )
    def body(x_hbm, emb_hbm, out_hbm, idx0, idx1, rows0, rows1,
             s_i0, s_i1, s_g, s_st0, s_st1):
        wid = lax.axis_index("s") * NC + lax.axis_index("c")
        base = wid * B_PER_W
        idx_v = (idx0, idx1)
        rows_v = (rows0, rows1)
        s_i = (s_i0, s_i1)
        s_st = (s_st0, s_st1)

        def idx_start(i, s):
            pltpu.async_copy(x_hbm.at[pl.ds(base + i * CHUNK, CHUNK)],
                             idx_v[s], s_i[s])

        def idx_wait(s):
            pltpu.make_async_copy(x_hbm.at[pl.ds(base, CHUNK)],
                                  idx_v[s], s_i[s]).wait()

        def gather(s):
            pltpu.async_copy(emb_hbm.at[idx_v[s]], rows_v[s], s_g).wait()

        def store_start(i, s):
            pltpu.async_copy(rows_v[s],
                             out_hbm.at[pl.ds(base + i * CHUNK, CHUNK)],
                             s_st[s])

        def store_wait(s):
            pltpu.make_async_copy(rows_v[s],
                                  out_hbm.at[pl.ds(base, CHUNK)],
                                  s_st[s]).wait()

        idx_start(0, 0)
        idx_wait(0)
        idx_start(1, 1)
        gather(0)
        idx_start(2, 0)
        store_start(0, 0)
        idx_wait(1)
        gather(1)
        idx_start(3, 1)
        store_start(1, 1)

        def half(i, s):
            idx_wait(s)
            store_wait(s)
            gather(s)

            @pl.when(i + 2 < N_CHUNKS)
            def _():
                idx_start(i + 2, s)

            store_start(i, s)

        def group(g, carry):
            i = 2 + 2 * g
            half(i, 0)
            half(i + 1, 1)
            return carry

        lax.fori_loop(0, (N_CHUNKS - 2) // 2, group, 0, unroll=False)

        store_wait(0)
        store_wait(1)

    return body(x_flat, emb)


def kernel(x, emb):
    x_flat = x.reshape(B).astype(jnp.int32)
    out = _sc_gather(x_flat, emb)
    return out.reshape(BATCH, HIST, D)


# speedup vs baseline: 1.7609x; 1.6403x over previous
import functools

import jax
import jax.numpy as jnp
from jax import lax
from jax.experimental import pallas as pl
from jax.experimental.pallas import tpu as pltpu
from jax.experimental.pallas import tpu_sc as plsc

BATCH = 16384
HIST = 200
D = 64
B = BATCH * HIST

_info = plsc.get_sparse_core_info()
NC, NS = _info.num_cores, _info.num_subcores
NW = NC * NS
B_PER_W = B // NW
CHUNK = 512
N_CHUNKS = B_PER_W // CHUNK

assert B % (8 * NW) == 0
assert B_PER_W % CHUNK == 0
assert N_CHUNKS % 2 == 0 and N_CHUNKS >= 4


def _sc_gather(x_flat, emb):
    mesh = plsc.VectorSubcoreMesh(core_axis_name="c", subcore_axis_name="s")

    @functools.partial(
        pl.kernel,
        mesh=mesh,
        out_type=jax.ShapeDtypeStruct((B, 2 * D), jnp.float32),
        scratch_types=[
            pltpu.VMEM((CHUNK,), jnp.int32),
            pltpu.VMEM((CHUNK,), jnp.int32),
            pltpu.VMEM((CHUNK, D), jnp.float32),
            pltpu.VMEM((CHUNK, D), jnp.float32),
            pltpu.SemaphoreType.DMA,
            pltpu.SemaphoreType.DMA,
            pltpu.SemaphoreType.DMA,
            pltpu.SemaphoreType.DMA,
            pltpu.SemaphoreType.DMA,
        ],
        compiler_params=pltpu.CompilerParams(use_tc_tiling_on_sc=False),
    )
    def body(x_hbm, emb_hbm, out_hbm, idx0, idx1, rows0, rows1,
             s_i0, s_i1, s_g, s_st0, s_st1):
        wid = lax.axis_index("s") * NC + lax.axis_index("c")
        base = wid * B_PER_W
        idx_v = (idx0, idx1)
        rows_v = (rows0, rows1)
        s_i = (s_i0, s_i1)
        s_st = (s_st0, s_st1)

        def idx_start(i, s):
            pltpu.async_copy(x_hbm.at[pl.ds(base + i * CHUNK, CHUNK)],
                             idx_v[s], s_i[s])

        def idx_wait(s):
            pltpu.make_async_copy(x_hbm.at[pl.ds(base, CHUNK)],
                                  idx_v[s], s_i[s]).wait()

        def gather(s):
            pltpu.async_copy(emb_hbm.at[idx_v[s]], rows_v[s], s_g).wait()

        def store_start(i, s):
            pltpu.async_copy(rows_v[s],
                             out_hbm.at[pl.ds(base + i * CHUNK, CHUNK),
                                        pl.ds(0, D)],
                             s_st[s])

        def store_wait(s):
            pltpu.make_async_copy(rows_v[s],
                                  out_hbm.at[pl.ds(base, CHUNK), pl.ds(0, D)],
                                  s_st[s]).wait()

        idx_start(0, 0)
        idx_wait(0)
        idx_start(1, 1)
        gather(0)
        idx_start(2, 0)
        store_start(0, 0)
        idx_wait(1)
        gather(1)
        idx_start(3, 1)
        store_start(1, 1)

        def half(i, s):
            idx_wait(s)
            store_wait(s)
            gather(s)

            @pl.when(i + 2 < N_CHUNKS)
            def _():
                idx_start(i + 2, s)

            store_start(i, s)

        def group(g, carry):
            i = 2 + 2 * g
            half(i, 0)
            half(i + 1, 1)
            return carry

        lax.fori_loop(0, (N_CHUNKS - 2) // 2, group, 0, unroll=False)

        store_wait(0)
        store_wait(1)

    return body(x_flat, emb)


def kernel(x, emb):
    x_flat = x.reshape(B).astype(jnp.int32)
    out = _sc_gather(x_flat, emb)
    return out[:, :D].reshape(BATCH, HIST, D)
